# baseline (speedup 1.0000x reference)
import functools

import jax
import jax.numpy as jnp
from jax import lax
from jax.experimental import pallas as pl
from jax.experimental.pallas import tpu as pltpu
from jax.experimental.pallas import tpu_sc as plsc

NC = 2
NS = 16
NW = NC * NS
L = 16


def _make_gather(B, H, V, D):
    bw = B // NW
    ng = bw // L
    mesh = plsc.VectorSubcoreMesh(core_axis_name="c", subcore_axis_name="s")

    @functools.partial(
        pl.kernel,
        mesh=mesh,
        out_type=jax.ShapeDtypeStruct((H, D, B), jnp.float32),
        scratch_types=[
            pltpu.VMEM((H, bw), jnp.int32),
            pltpu.VMEM((4, bw, D), jnp.float32),
            pltpu.VMEM((2, D, bw), jnp.float32),
            pltpu.SemaphoreType.DMA,
            pltpu.SemaphoreType.DMA,
        ],
        compiler_params=pltpu.CompilerParams(
            use_tc_tiling_on_sc=False, needs_layout_passes=False
        ),
    )
    def k(idx_hbm, table_hbm, out_hbm, idx_v, buf, bt, sem_g, sem_s):
        wid = lax.axis_index("s") * NC + lax.axis_index("c")
        col0 = wid * bw
        pltpu.sync_copy(idx_hbm.at[:, pl.ds(col0, bw)], idx_v)

        rows = [lax.iota(jnp.int32, L) + (g * L) for g in range(ng)]
        cols = [jnp.full((L,), d, jnp.int32) for d in range(D)]

        def fire(h, p):
            pltpu.async_copy(table_hbm.at[idx_v.at[h]], buf.at[p], sem_g)

        def wait_gather(p):
            pltpu.make_async_copy(
                table_hbm.at[pl.ds(0, bw)], buf.at[p], sem_g
            ).wait()

        def transpose(p, q):
            for d in range(D):
                for g in range(ng):
                    v = plsc.load_gather(buf.at[p], [rows[g], cols[d]])
                    bt[q, d, pl.ds(g * L, L)] = v

        def store(h, q):
            pltpu.async_copy(
                bt.at[q], out_hbm.at[h, :, pl.ds(col0, bw)], sem_s
            )

        def wait_store(q):
            pltpu.make_async_copy(
                bt.at[q], out_hbm.at[0, :, pl.ds(col0, bw)], sem_s
            ).wait()

        for p in range(4):
            fire(p, p)

        def body(i, carry):
            h0 = i * 4
            for p in range(4):
                q = p % 2
                wait_gather(p)

                if p >= 2:
                    wait_store(q)
                else:

                    @pl.when(i > 0)
                    def _():
                        wait_store(q)

                transpose(p, q)

                @pl.when(i < H // 4 - 1)
                def _():
                    fire(h0 + 4 + p, p)

                store(h0 + p, q)
            return carry

        lax.fori_loop(0, H // 4, body, 0, unroll=False)
        wait_store(0)
        wait_store(1)

    return k


def kernel(x, W):
    B, H = x.shape
    V, D = W.shape
    out_t = _make_gather(B, H, V, D)(x.T.astype(jnp.int32), W)
    return jnp.transpose(out_t, (2, 0, 1))


# speedup vs baseline: 1.1891x; 1.1891x over previous
import functools

import jax
import jax.numpy as jnp
from jax import lax
from jax.experimental import pallas as pl
from jax.experimental.pallas import tpu as pltpu
from jax.experimental.pallas import tpu_sc as plsc

NC = 2
NS = 16
NW = NC * NS
L = 16


def _make_gather(B, H, V, D):
    bw = B // NW
    ng = bw // L
    mesh = plsc.VectorSubcoreMesh(core_axis_name="c", subcore_axis_name="s")

    @functools.partial(
        pl.kernel,
        mesh=mesh,
        out_type=jax.ShapeDtypeStruct((H, D, B), jnp.float32),
        scratch_types=[
            pltpu.VMEM((H, bw), jnp.int32),
            pltpu.VMEM((4, bw, D), jnp.float32),
            pltpu.VMEM((2, D * bw + 8), jnp.float32),
            pltpu.SemaphoreType.DMA,
            pltpu.SemaphoreType.DMA,
        ],
        compiler_params=pltpu.CompilerParams(
            use_tc_tiling_on_sc=False, needs_layout_passes=False
        ),
    )
    def k(idx_hbm, table_hbm, out_hbm, idx_v, buf, bt, sem_g, sem_s):
        wid = lax.axis_index("s") * NC + lax.axis_index("c")
        col0 = wid * bw
        pltpu.sync_copy(idx_hbm.at[:, pl.ds(col0, bw)], idx_v)

        idx_lo = lax.iota(jnp.int32, L) * bw
        idx_hi = (lax.iota(jnp.int32, L) + L) * bw
        span = (D - 1) * bw + 8

        def fire(h, p):
            pltpu.async_copy(table_hbm.at[idx_v.at[h]], buf.at[p], sem_g)

        def wait_gather(p):
            pltpu.make_async_copy(
                table_hbm.at[pl.ds(0, bw)], buf.at[p], sem_g
            ).wait()

        idx_k = [(idx_lo + k, idx_hi + k) for k in range(8)]

        def transpose(p, q):
            def body(c8, carry):
                c0 = pl.multiple_of(c8 * 8, 8)
                sl = bt.at[q, pl.ds(c0, span)]
                for k in range(8):
                    v_lo = buf[p, c0 + k, pl.ds(0, L)]
                    v_hi = buf[p, c0 + k, pl.ds(L, L)]
                    plsc.store_scatter(sl, [idx_k[k][0]], v_lo)
                    plsc.store_scatter(sl, [idx_k[k][1]], v_hi)
                return carry

            lax.fori_loop(0, bw // 8, body, 0, unroll=2)

        def store(h, q):
            for d in range(D):
                pltpu.async_copy(
                    bt.at[q, pl.ds(d * bw, bw)],
                    out_hbm.at[h, d, pl.ds(col0, bw)],
                    sem_s,
                )

        def wait_store(q):
            for d in range(D):
                pltpu.make_async_copy(
                    bt.at[q, pl.ds(d * bw, bw)],
                    out_hbm.at[0, 0, pl.ds(col0, bw)],
                    sem_s,
                ).wait()

        for p in range(4):
            fire(p, p)

        def body(i, carry):
            h0 = i * 4
            for p in range(4):
                q = p % 2
                wait_gather(p)

                if p >= 2:
                    wait_store(q)
                else:

                    @pl.when(i > 0)
                    def _():
                        wait_store(q)

                transpose(p, q)

                @pl.when(i < H // 4 - 1)
                def _():
                    fire(h0 + 4 + p, p)

                store(h0 + p, q)
            return carry

        lax.fori_loop(0, H // 4, body, 0, unroll=False)
        wait_store(0)
        wait_store(1)

    return k


def kernel(x, W):
    B, H = x.shape
    V, D = W.shape
    out_t = _make_gather(B, H, V, D)(x.T.astype(jnp.int32), W)
    return jnp.transpose(out_t, (2, 0, 1))


# speedup vs baseline: 1.2179x; 1.0242x over previous
import functools

import jax
import jax.numpy as jnp
from jax import lax
from jax.experimental import pallas as pl
from jax.experimental.pallas import tpu as pltpu
from jax.experimental.pallas import tpu_sc as plsc

NC = 2
NS = 16
NW = NC * NS
L = 16


def _make_gather(B, H, V, D):
    bw = B // NW
    ng = bw // L
    mesh = plsc.VectorSubcoreMesh(core_axis_name="c", subcore_axis_name="s")

    @functools.partial(
        pl.kernel,
        mesh=mesh,
        out_type=jax.ShapeDtypeStruct((H, D, B), jnp.float32),
        scratch_types=[
            pltpu.VMEM((H, bw), jnp.int32),
            pltpu.VMEM((4, bw, D), jnp.float32),
            pltpu.VMEM((2, D, bw), jnp.float32),
            pltpu.SemaphoreType.DMA,
            pltpu.SemaphoreType.DMA,
        ],
        compiler_params=pltpu.CompilerParams(
            use_tc_tiling_on_sc=False, needs_layout_passes=False
        ),
    )
    def k(idx_hbm, table_hbm, out_hbm, idx_v, buf, bt, sem_g, sem_s):
        wid = lax.axis_index("s") * NC + lax.axis_index("c")
        col0 = wid * bw
        pltpu.sync_copy(idx_hbm.at[:, pl.ds(col0, bw)], idx_v)

        rows_lo = lax.iota(jnp.int32, L)
        rows_hi = rows_lo + L

        def fire(h, p):
            pltpu.async_copy(table_hbm.at[idx_v.at[h]], buf.at[p], sem_g)

        def wait_gather(p):
            pltpu.make_async_copy(
                table_hbm.at[pl.ds(0, bw)], buf.at[p], sem_g
            ).wait()

        def transpose(p, q):
            def body(c, carry):
                cols = jnp.full((L,), c, jnp.int32)
                v_lo = buf[p, c, pl.ds(0, L)]
                v_hi = buf[p, c, pl.ds(L, L)]
                plsc.store_scatter(bt.at[q], [rows_lo, cols], v_lo)
                plsc.store_scatter(bt.at[q], [rows_hi, cols], v_hi)
                return carry

            lax.fori_loop(0, bw, body, 0, unroll=8)

        def store(h, q):
            pltpu.async_copy(
                bt.at[q], out_hbm.at[h, :, pl.ds(col0, bw)], sem_s
            )

        def wait_store(q):
            pltpu.make_async_copy(
                bt.at[q], out_hbm.at[0, :, pl.ds(col0, bw)], sem_s
            ).wait()

        for p in range(4):
            fire(p, p)

        def body(i, carry):
            h0 = i * 4
            for p in range(4):
                q = p % 2
                wait_gather(p)

                if p >= 2:
                    wait_store(q)
                else:

                    @pl.when(i > 0)
                    def _():
                        wait_store(q)

                transpose(p, q)

                @pl.when(i < H // 4 - 1)
                def _():
                    fire(h0 + 4 + p, p)

                store(h0 + p, q)
            return carry

        lax.fori_loop(0, H // 4, body, 0, unroll=False)
        wait_store(0)
        wait_store(1)

    return k


def kernel(x, W):
    B, H = x.shape
    V, D = W.shape
    out_t = _make_gather(B, H, V, D)(x.T.astype(jnp.int32), W)
    return jnp.transpose(out_t, (2, 0, 1))


# speedup vs baseline: 1.4909x; 1.2242x over previous
import functools

import jax
import jax.numpy as jnp
from jax import lax
from jax.experimental import pallas as pl
from jax.experimental.pallas import tpu as pltpu
from jax.experimental.pallas import tpu_sc as plsc

NC = 2
NS = 16
NW = NC * NS
L = 16


def _make_gather(B, H, V, D):
    bw = B // NW
    ng = bw // L
    mesh = plsc.VectorSubcoreMesh(core_axis_name="c", subcore_axis_name="s")

    @functools.partial(
        pl.kernel,
        mesh=mesh,
        out_type=jax.ShapeDtypeStruct((H, D, B), jnp.float32),
        scratch_types=[
            pltpu.VMEM((H, bw), jnp.int32),
            pltpu.VMEM((4, bw, D), jnp.float32),
            pltpu.VMEM((2, D, bw + 9), jnp.float32),
            pltpu.SemaphoreType.DMA,
            pltpu.SemaphoreType.DMA,
        ],
        compiler_params=pltpu.CompilerParams(
            use_tc_tiling_on_sc=False, needs_layout_passes=False
        ),
    )
    def k(idx_hbm, table_hbm, out_hbm, idx_v, buf, bt, sem_g, sem_s):
        wid = lax.axis_index("s") * NC + lax.axis_index("c")
        col0 = wid * bw
        pltpu.sync_copy(idx_hbm.at[:, pl.ds(col0, bw)], idx_v)

        rows_lo = lax.iota(jnp.int32, L)
        rows_hi = rows_lo + L

        def fire(h, p):
            pltpu.async_copy(table_hbm.at[idx_v.at[h]], buf.at[p], sem_g)

        def wait_gather(p):
            pltpu.make_async_copy(
                table_hbm.at[pl.ds(0, bw)], buf.at[p], sem_g
            ).wait()

        def transpose(p, q):
            def body(c, carry):
                cols = jnp.full((L,), c, jnp.int32)
                v_lo = buf[p, c, pl.ds(0, L)]
                v_hi = buf[p, c, pl.ds(L, L)]
                plsc.store_scatter(bt.at[q], [rows_lo, cols], v_lo)
                plsc.store_scatter(bt.at[q], [rows_hi, cols], v_hi)
                return carry

            lax.fori_loop(0, bw, body, 0, unroll=8)

        def store(h, q):
            pltpu.async_copy(
                bt.at[q, :, pl.ds(0, bw)],
                out_hbm.at[h, :, pl.ds(col0, bw)],
                sem_s,
            )

        def wait_store(q):
            pltpu.make_async_copy(
                bt.at[q, :, pl.ds(0, bw)],
                out_hbm.at[0, :, pl.ds(col0, bw)],
                sem_s,
            ).wait()

        for p in range(4):
            fire(p, p)

        def body(i, carry):
            h0 = i * 4
            for p in range(4):
                q = p % 2
                wait_gather(p)

                if p >= 2:
                    wait_store(q)
                else:

                    @pl.when(i > 0)
                    def _():
                        wait_store(q)

                transpose(p, q)

                @pl.when(i < H // 4 - 1)
                def _():
                    fire(h0 + 4 + p, p)

                store(h0 + p, q)
            return carry

        lax.fori_loop(0, H // 4, body, 0, unroll=False)
        wait_store(0)
        wait_store(1)

    return k


def kernel(x, W):
    B, H = x.shape
    V, D = W.shape
    out_t = _make_gather(B, H, V, D)(x.T.astype(jnp.int32), W)
    return jnp.transpose(out_t, (2, 0, 1))
